# baseline (speedup 1.0000x reference)
import jax
import jax.numpy as jnp
from jax import lax
from jax.experimental import pallas as pl
from jax.experimental.pallas import tpu as pltpu
from jax.experimental.pallas import tpu_sc as plsc

NUM_CORES = 2
NUM_SUBCORES = 16
LANES = 16
NW = NUM_CORES * NUM_SUBCORES

IDX_CHUNK = 128


def _make_sc_kernel(B, D, E, S):
    assert B % NW == 0
    bpw = B // NW
    n_chunks = bpw // IDX_CHUNK
    n_groups = bpw // LANES
    mesh = plsc.VectorSubcoreMesh(core_axis_name="c", subcore_axis_name="s")

    def body(pairs_hbm, ew_hbm, sw_hbm, out_hbm,
             eflat_v, eidx_v, sidx_v, erows_v, stab_v, out_v, sem):
        wid = lax.axis_index("s") * NUM_CORES + lax.axis_index("c")
        base = wid * bpw
        lane = lax.iota(jnp.int32, LANES)

        stab_cp = pltpu.async_copy(sw_hbm, stab_v, sem)

        pltpu.sync_copy(pairs_hbm.at[0, pl.ds(base, bpw)], eflat_v)
        pltpu.sync_copy(pairs_hbm.at[1, pl.ds(base, bpw)], sidx_v)

        for c in range(bpw // LANES):
            sl = pl.ds(c * LANES, LANES)
            ei = jnp.minimum(jnp.maximum(eflat_v[sl], 0), E - 1)
            row, off = divmod(c * LANES, IDX_CHUNK)
            eidx_v[row, pl.ds(off, LANES)] = ei
            sidx_v[sl] = jnp.minimum(jnp.maximum(sidx_v[sl], 0), S - 1)

        copies = []
        for j in range(n_chunks):
            dst = pl.ds(j * IDX_CHUNK, IDX_CHUNK)
            copies.append(pltpu.async_copy(ew_hbm.at[eidx_v.at[j]], erows_v.at[dst], sem))
        for cp in copies:
            cp.wait()
        stab_cp.wait()

        def g_body(g, carry):
            r = g * LANES + lane
            sid = sidx_v[pl.ds(g * LANES, LANES)]
            acc = jnp.zeros((LANES,), jnp.float32)
            for d in range(D):
                dv = jnp.full((LANES,), d, jnp.int32)
                ev = plsc.load_gather(erows_v, [r, dv])
                sv = plsc.load_gather(stab_v, [dv, sid])
                acc = acc + ev * sv
            out_v[pl.ds(g * LANES, LANES)] = 1.0 / (1.0 + jnp.exp(-acc))
            return carry

        lax.fori_loop(0, n_groups, g_body, 0)
        pltpu.sync_copy(out_v, out_hbm.at[pl.ds(base, bpw)])

    return pl.kernel(
        body,
        out_type=jax.ShapeDtypeStruct((B,), jnp.float32),
        mesh=mesh,
        compiler_params=pltpu.CompilerParams(
            needs_layout_passes=False, use_tc_tiling_on_sc=False),
        scratch_types=[
            pltpu.VMEM((bpw,), jnp.int32),
            pltpu.VMEM((n_chunks, IDX_CHUNK), jnp.int32),
            pltpu.VMEM((bpw,), jnp.int32),
            pltpu.VMEM((bpw, D), jnp.float32),
            pltpu.VMEM((D, S), jnp.float32),
            pltpu.VMEM((bpw,), jnp.float32),
            pltpu.SemaphoreType.DMA,
        ],
    )


def kernel(pairs, exercise_w, skill_w):
    B = pairs.shape[0]
    E, D = exercise_w.shape
    S = skill_w.shape[0]
    sc = _make_sc_kernel(B, D, E, S)
    return sc(pairs.T, exercise_w, skill_w.T)
